# Initial kernel scaffold; baseline (speedup 1.0000x reference)
import jax
import jax.numpy as jnp
from jax.experimental import pallas as pl


def kernel(item_ids, item_actions, item_emb_table, ratio_emb_table):
    raise NotImplementedError("write your pallas kernel here")


# speedup vs baseline: 1.3005x; 1.3005x over previous
import functools

import jax
import jax.numpy as jnp
from jax import lax
from jax.experimental import pallas as pl
from jax.experimental.pallas import tpu as pltpu
from jax.experimental.pallas import tpu_sc as plsc

_B, _N, _D = 4096, 50, 128
_TOTAL = _B * _N
_NC, _NS = 2, 16
_NW = _NC * _NS
_PER_W = _TOTAL // _NW
_CHUNK = 320
_NCHUNK = _PER_W // _CHUNK


def _body(ids_hbm, act_hbm, table_hbm, ratio_hbm, out_hbm,
          idx_v, act_v, buf, gsem):
    wid = lax.axis_index("s") * _NC + lax.axis_index("c")
    w_base = wid * _PER_W

    def chunk(c, carry):
        base = w_base + c * _CHUNK
        pltpu.sync_copy(ids_hbm.at[pl.ds(base, _CHUNK)], idx_v)
        pltpu.sync_copy(act_hbm.at[pl.ds(base, _CHUNK)], act_v)
        pltpu.async_copy(table_hbm.at[idx_v], buf, gsem).wait()
        pltpu.sync_copy(buf, out_hbm.at[pl.ds(base, _CHUNK), 0])
        pltpu.async_copy(ratio_hbm.at[act_v], buf, gsem, add=True).wait()
        pltpu.sync_copy(buf, out_hbm.at[pl.ds(base, _CHUNK), 1])
        return carry

    lax.fori_loop(0, _NCHUNK, chunk, 0)


@jax.jit
def _run(ids_flat, act_flat, item_emb_table, ratio_emb_table):
    mesh = plsc.VectorSubcoreMesh(core_axis_name="c", subcore_axis_name="s")
    k = pl.kernel(
        _body,
        out_type=jax.ShapeDtypeStruct((_TOTAL, 2, _D), jnp.float32),
        mesh=mesh,
        scratch_types=[
            pltpu.VMEM((_CHUNK,), jnp.int32),
            pltpu.VMEM((_CHUNK,), jnp.int32),
            pltpu.VMEM((_CHUNK, _D), jnp.float32),
            pltpu.SemaphoreType.DMA,
        ],
    )
    return k(ids_flat, act_flat, item_emb_table, ratio_emb_table)


def kernel(item_ids, item_actions, item_emb_table, ratio_emb_table):
    ids_flat = item_ids.reshape(_TOTAL).astype(jnp.int32)
    act_flat = item_actions.reshape(_TOTAL).astype(jnp.int32)
    out = _run(ids_flat, act_flat,
               item_emb_table.astype(jnp.float32),
               ratio_emb_table.astype(jnp.float32))
    return out.reshape(_B, 2 * _N, _D)
